# baseline (speedup 1.0000x reference)
import functools
import math

import jax
import jax.numpy as jnp
from jax import lax
from jax.experimental import pallas as pl
from jax.experimental.pallas import tpu as pltpu
from jax.experimental.pallas import tpu_sc as plsc

N = 10000
E = 320000
IN_DIM = 144
HID = 64

NC = 2
NS = 16
TILES = NC * NS
EPT = E // TILES
CHUNK = 80
NCHUNKS = EPT // CHUNK
NBUF = 5
NGROUPS = NCHUNKS // NBUF
RSTRIDE = 624
RLEN = 640

_MESH = dict(
    mesh=plsc.VectorSubcoreMesh(core_axis_name="c", subcore_axis_name="s"),
    compiler_params=pltpu.CompilerParams(
        use_tc_tiling_on_sc=False, needs_layout_passes=False),
)


def _fill(ref, shape, value):
    if len(shape) == 1:
        for j in range(shape[0] // 16):
            ref[pl.ds(j * 16, 16)] = jnp.full((16,), value, jnp.float32)
    else:
        for r in range(shape[0]):
            for j in range(shape[1] // 16):
                ref[r, pl.ds(j * 16, 16)] = jnp.full((16,), value, jnp.float32)


def _zero_acc_slice(zbuf, acc_sh, r0, zsem, rows):
    nrep = RLEN // rows
    for j in range(nrep):
        pltpu.async_copy(zbuf, acc_sh.at[pl.ds(r0 + j * rows, rows)], zsem)
    for j in range(nrep):
        pltpu.make_async_copy(zbuf, acc_sh.at[pl.ds(r0, rows)], zsem).wait()


@functools.partial(
    pl.kernel,
    out_type=jax.ShapeDtypeStruct((NC * N,), jnp.float32),
    scratch_types=[
        pltpu.VMEM((EPT,), jnp.int32),
        pltpu.VMEM((NBUF, CHUNK), jnp.int32),
        pltpu.VMEM((CHUNK,), jnp.float32),
        pltpu.VMEM((CHUNK,), jnp.float32),
        pltpu.VMEM((RLEN,), jnp.float32),
        pltpu.VMEM_SHARED((N,), jnp.float32),
        pltpu.SemaphoreType.DMA,
        pltpu.SemaphoreType.DMA((NBUF,)),
    ],
    **_MESH,
)
def _sc_deg(epk_hbm, out_hbm, pk_b, didx_p, ones_v, zv, buf_v, acc_sh,
            zsem, ssem):
    cid = lax.axis_index("c")
    sid = lax.axis_index("s")
    t = cid * NS + sid
    r0 = sid * RSTRIDE
    pltpu.sync_copy(epk_hbm.at[pl.ds(t * EPT, EPT)], pk_b)
    _fill(ones_v, (CHUNK,), 1.0)
    _fill(zv, (CHUNK,), 0.0)
    _zero_acc_slice(zv, acc_sh, r0, zsem, CHUNK)
    plsc.subcore_barrier()

    def _unpack_dst(c, b):
        for j in range(CHUNK // 16):
            pk = pk_b[pl.ds(c * CHUNK + j * 16, 16)]
            didx_p[b, pl.ds(j * 16, 16)] = lax.bitwise_and(pk, 16383)

    def _start(b):
        pltpu.async_copy(ones_v, acc_sh.at[didx_p.at[b]], ssem.at[b], add=True)

    def _wait(b):
        pltpu.make_async_copy(
            ones_v, acc_sh.at[didx_p.at[b]], ssem.at[b]).wait()

    for b in range(NBUF):
        _unpack_dst(b, b)
        _start(b)

    def body(k, carry):
        for b in range(NBUF):
            _wait(b)
            _unpack_dst(k * NBUF + b, b)
            _start(b)
        return carry

    lax.fori_loop(1, NGROUPS, body, 0)
    for b in range(NBUF):
        _wait(b)
    plsc.subcore_barrier()
    pltpu.sync_copy(acc_sh.at[pl.ds(r0, RLEN)], buf_v)
    pltpu.sync_copy(buf_v, out_hbm.at[pl.ds(cid * N + r0, RLEN)])


@functools.partial(
    pl.kernel,
    out_type=jax.ShapeDtypeStruct((NC * N, HID), jnp.float32),
    scratch_types=[
        pltpu.VMEM((EPT,), jnp.int32),
        pltpu.VMEM((NBUF, CHUNK), jnp.int32),
        pltpu.VMEM((NBUF, CHUNK), jnp.int32),
        pltpu.VMEM((NBUF, CHUNK, HID), jnp.float32),
        pltpu.VMEM((CHUNK, HID), jnp.float32),
        pltpu.VMEM_SHARED((N, HID), jnp.float32),
        pltpu.SemaphoreType.DMA,
        pltpu.SemaphoreType.DMA((NBUF,)),
        pltpu.SemaphoreType.DMA((NBUF,)),
    ],
    **_MESH,
)
def _sc_conv1(epk_hbm, g1_hbm, out_hbm,
              pk_b, sidx_p, didx_p, rows, zrow, acc_sh, zsem, gsem, ssem):
    cid = lax.axis_index("c")
    sid = lax.axis_index("s")
    t = cid * NS + sid
    r0 = sid * RSTRIDE
    pltpu.sync_copy(epk_hbm.at[pl.ds(t * EPT, EPT)], pk_b)
    _fill(zrow, (CHUNK, HID), 0.0)
    _zero_acc_slice(zrow, acc_sh, r0, zsem, CHUNK)
    plsc.subcore_barrier()

    def _unpack(c, b):
        for j in range(CHUNK // 16):
            pk = pk_b[pl.ds(c * CHUNK + j * 16, 16)]
            sidx_p[b, pl.ds(j * 16, 16)] = lax.shift_right_logical(pk, 14)
            didx_p[b, pl.ds(j * 16, 16)] = lax.bitwise_and(pk, 16383)

    def _gstart(b):
        pltpu.async_copy(g1_hbm.at[sidx_p.at[b]], rows.at[b], gsem.at[b])

    def _gwait(b):
        pltpu.make_async_copy(
            g1_hbm.at[sidx_p.at[b]], rows.at[b], gsem.at[b]).wait()

    def _sstart(b):
        pltpu.async_copy(
            rows.at[b], acc_sh.at[didx_p.at[b]], ssem.at[b], add=True)

    def _swait(b):
        pltpu.make_async_copy(
            rows.at[b], acc_sh.at[didx_p.at[b]], ssem.at[b]).wait()

    for b in range(NBUF):
        _unpack(b, b)
        _gstart(b)

    def body(k, carry):
        for b in range(NBUF):
            _gwait(b)
            _sstart(b)
        for b in range(NBUF):
            _swait(b)
            _unpack(k * NBUF + b, b)
            _gstart(b)
        return carry

    lax.fori_loop(1, NGROUPS, body, 0)
    for b in range(NBUF):
        _gwait(b)
        _sstart(b)
    for b in range(NBUF):
        _swait(b)
    plsc.subcore_barrier()
    for j in range(RLEN // CHUNK):
        b = j % NBUF
        if j >= NBUF:
            pltpu.make_async_copy(
                rows.at[b], out_hbm.at[pl.ds(cid * N + r0, CHUNK)],
                ssem.at[b]).wait()
        pltpu.async_copy(
            acc_sh.at[pl.ds(r0 + j * CHUNK, CHUNK)], rows.at[b], gsem.at[b])
        pltpu.make_async_copy(
            acc_sh.at[pl.ds(r0, CHUNK)], rows.at[b], gsem.at[b]).wait()
        pltpu.async_copy(
            rows.at[b], out_hbm.at[pl.ds(cid * N + r0 + j * CHUNK, CHUNK)],
            ssem.at[b])
    for j in range(NBUF):
        pltpu.make_async_copy(
            rows.at[j], out_hbm.at[pl.ds(cid * N + r0, CHUNK)],
            ssem.at[j]).wait()


@functools.partial(
    pl.kernel,
    out_type=jax.ShapeDtypeStruct((NC * N,), jnp.float32),
    scratch_types=[
        pltpu.VMEM((N,), jnp.float32),
        pltpu.VMEM((EPT,), jnp.int32),
        pltpu.VMEM((NBUF, CHUNK), jnp.int32),
        pltpu.VMEM((NBUF, CHUNK), jnp.float32),
        pltpu.VMEM((CHUNK,), jnp.float32),
        pltpu.VMEM((RLEN,), jnp.float32),
        pltpu.VMEM_SHARED((N,), jnp.float32),
        pltpu.SemaphoreType.DMA,
        pltpu.SemaphoreType.DMA((NBUF,)),
    ],
    **_MESH,
)
def _sc_conv2(epk_hbm, g2_hbm, out_hbm,
              g2_v, pk_b, didx_p, vals, zv, buf_v, acc_sh, zsem, ssem):
    cid = lax.axis_index("c")
    sid = lax.axis_index("s")
    t = cid * NS + sid
    r0 = sid * RSTRIDE
    pltpu.sync_copy(epk_hbm.at[pl.ds(t * EPT, EPT)], pk_b)
    pltpu.sync_copy(g2_hbm, g2_v)
    _fill(zv, (CHUNK,), 0.0)
    _zero_acc_slice(zv, acc_sh, r0, zsem, CHUNK)
    plsc.subcore_barrier()

    def _chunk(c, b):
        for j in range(CHUNK // 16):
            pk = pk_b[pl.ds(c * CHUNK + j * 16, 16)]
            sv = lax.shift_right_logical(pk, 14)
            didx_p[b, pl.ds(j * 16, 16)] = lax.bitwise_and(pk, 16383)
            vals[b, pl.ds(j * 16, 16)] = plsc.load_gather(g2_v, [sv])
        pltpu.async_copy(
            vals.at[b], acc_sh.at[didx_p.at[b]], ssem.at[b], add=True)

    def _swait(b):
        pltpu.make_async_copy(
            vals.at[b], acc_sh.at[didx_p.at[b]], ssem.at[b]).wait()

    for b in range(NBUF):
        _chunk(b, b)

    def body(k, carry):
        for b in range(NBUF):
            _swait(b)
            _chunk(k * NBUF + b, b)
        return carry

    lax.fori_loop(1, NGROUPS, body, 0)
    for b in range(NBUF):
        _swait(b)
    plsc.subcore_barrier()
    pltpu.sync_copy(acc_sh.at[pl.ds(r0, RLEN)], buf_v)
    pltpu.sync_copy(buf_v, out_hbm.at[pl.ds(cid * N + r0, RLEN)])


NP = N // 2


def _dense1_body(xp_ref, w1p_ref, cnt2_ref, s_ref, g1p_ref, dinv2_ref):
    deg2 = cnt2_ref[0] + cnt2_ref[1] + 1.0
    dinv2 = lax.rsqrt(deg2)
    dp = jnp.dot(dinv2, s_ref[...], preferred_element_type=jnp.float32)
    hp = jnp.dot(xp_ref[...], w1p_ref[...], preferred_element_type=jnp.float32)
    g1p_ref[...] = hp * dp
    dinv2_ref[...] = dinv2


def _dense2_body(p2_ref, g1p_ref, dinv2_ref, s_ref, w2p_ref, b1p_ref,
                 g2_ref):
    s1p = p2_ref[pl.ds(0, NP), :] + p2_ref[pl.ds(NP, NP), :] + g1p_ref[...]
    dinv2 = dinv2_ref[...]
    dp = jnp.dot(dinv2, s_ref[...], preferred_element_type=jnp.float32)
    o1p = s1p * dp + b1p_ref[...][None, :]
    ap = 0.5 * o1p * (1.0 + lax.erf(o1p * (1.0 / math.sqrt(2.0))))
    h2 = jnp.dot(ap, w2p_ref[...], preferred_element_type=jnp.float32)
    g2_ref[...] = h2 * dinv2


def _head_body(q2_ref, g2_ref, dinv2_ref, s_ref, b2_ref, mw1p_ref,
               mb1p_ref, mw2p_ref, mb2_ref, out_ref):
    s2 = q2_ref[0] + q2_ref[1] + g2_ref[...]
    o2 = s2 * dinv2_ref[...] + b2_ref[0]
    o2p = jnp.dot(o2, s_ref[...], preferred_element_type=jnp.float32)
    t = o2p * mw1p_ref[...][None, :] + mb1p_ref[...][None, :]
    r = jnp.maximum(t, 0.0)
    y = jnp.dot(r, mw2p_ref[...], preferred_element_type=jnp.float32)
    out_ref[...] = y + mb2_ref[0]


_dense1 = pl.pallas_call(
    _dense1_body,
    out_shape=[
        jax.ShapeDtypeStruct((NP, 2 * HID), jnp.float32),
        jax.ShapeDtypeStruct((NP, 2), jnp.float32),
    ],
)

_dense2 = pl.pallas_call(
    _dense2_body,
    out_shape=jax.ShapeDtypeStruct((NP, 2), jnp.float32),
)

_head = pl.pallas_call(
    _head_body,
    out_shape=jax.ShapeDtypeStruct((NP, 2), jnp.float32),
)


def kernel(x, edge_index, W1, b1, W2, b2, mW1, mb1, mW2, mb2):
    xp = x.reshape(NP, 2 * IN_DIM)
    epk = jnp.left_shift(edge_index[0], 14) | edge_index[1]

    w1p = jnp.zeros((2 * IN_DIM, 2 * HID), jnp.float32)
    w1p = w1p.at[:IN_DIM, :HID].set(W1).at[IN_DIM:, HID:].set(W1)
    sel = jnp.zeros((2, 2 * HID), jnp.float32)
    sel = sel.at[0, :HID].set(1.0).at[1, HID:].set(1.0)
    b1p = jnp.concatenate([b1, b1])
    w2p = jnp.zeros((2 * HID, 2), jnp.float32)
    w2p = w2p.at[:HID, 0].set(W2[:, 0]).at[HID:, 1].set(W2[:, 0])
    mw1p = jnp.concatenate([mW1[0], mW1[0]])
    mb1p = jnp.concatenate([mb1, mb1])
    mw2p = jnp.zeros((2 * HID, 2), jnp.float32)
    mw2p = mw2p.at[:HID, 0].set(mW2[:, 0]).at[HID:, 1].set(mW2[:, 0])

    cnt2 = _sc_deg(epk).reshape(NC, NP, 2)
    g1p, dinv2 = _dense1(xp, w1p, cnt2, sel)
    p2 = _sc_conv1(epk, g1p.reshape(N, HID)).reshape(NC * NP, 2 * HID)
    g2_2 = _dense2(p2, g1p, dinv2, sel, w2p, b1p)
    q2 = _sc_conv2(epk, g2_2.reshape(N)).reshape(NC, NP, 2)
    y2 = _head(q2, g2_2, dinv2, sel, b2, mw1p, mb1p, mw2p, mb2)
    return y2.reshape(1, 1, N, 1)


# speedup vs baseline: 1.2151x; 1.2151x over previous
import functools
import math

import jax
import jax.numpy as jnp
from jax import lax
from jax.experimental import pallas as pl
from jax.experimental.pallas import tpu as pltpu
from jax.experimental.pallas import tpu_sc as plsc

N = 10000
E = 320000
IN_DIM = 144
HID = 64

NC = 2
NS = 16
TILES = NC * NS
EPT = E // TILES
CHUNK = 80
NCHUNKS = EPT // CHUNK
NBUF = 5
NGROUPS = NCHUNKS // NBUF
RSTRIDE = 624
RLEN = 640

_MESH = dict(
    mesh=plsc.VectorSubcoreMesh(core_axis_name="c", subcore_axis_name="s"),
    compiler_params=pltpu.CompilerParams(
        use_tc_tiling_on_sc=False, needs_layout_passes=False),
)


def _fill(ref, shape, value):
    if len(shape) == 1:
        for j in range(shape[0] // 16):
            ref[pl.ds(j * 16, 16)] = jnp.full((16,), value, jnp.float32)
    else:
        for r in range(shape[0]):
            for j in range(shape[1] // 16):
                ref[r, pl.ds(j * 16, 16)] = jnp.full((16,), value, jnp.float32)


def _zero_acc_slice(zbuf, acc_sh, r0, zsem, rows):
    nrep = RLEN // rows
    for j in range(nrep):
        pltpu.async_copy(zbuf, acc_sh.at[pl.ds(r0 + j * rows, rows)], zsem)
    for j in range(nrep):
        pltpu.make_async_copy(zbuf, acc_sh.at[pl.ds(r0, rows)], zsem).wait()


@functools.partial(
    pl.kernel,
    out_type=jax.ShapeDtypeStruct((NC * N,), jnp.float32),
    scratch_types=[
        pltpu.VMEM((EPT,), jnp.int32),
        pltpu.VMEM((NBUF, CHUNK), jnp.int32),
        pltpu.VMEM((CHUNK,), jnp.float32),
        pltpu.VMEM((CHUNK,), jnp.float32),
        pltpu.VMEM((RLEN,), jnp.float32),
        pltpu.VMEM_SHARED((N,), jnp.float32),
        pltpu.SemaphoreType.DMA,
        pltpu.SemaphoreType.DMA((NBUF,)),
    ],
    **_MESH,
)
def _sc_deg(epk_hbm, out_hbm, pk_b, didx_p, ones_v, zv, buf_v, acc_sh,
            zsem, ssem):
    cid = lax.axis_index("c")
    sid = lax.axis_index("s")
    t = cid * NS + sid
    r0 = sid * RSTRIDE
    pltpu.sync_copy(epk_hbm.at[pl.ds(t * EPT, EPT)], pk_b)
    _fill(ones_v, (CHUNK,), 1.0)
    _fill(zv, (CHUNK,), 0.0)
    _zero_acc_slice(zv, acc_sh, r0, zsem, CHUNK)
    plsc.subcore_barrier()

    def _unpack_dst(c, b):
        for j in range(CHUNK // 16):
            pk = pk_b[pl.ds(c * CHUNK + j * 16, 16)]
            didx_p[b, pl.ds(j * 16, 16)] = lax.bitwise_and(pk, 16383)

    def _start(b):
        pltpu.async_copy(ones_v, acc_sh.at[didx_p.at[b]], ssem.at[b], add=True)

    def _wait(b):
        pltpu.make_async_copy(
            ones_v, acc_sh.at[didx_p.at[b]], ssem.at[b]).wait()

    for b in range(NBUF):
        _unpack_dst(b, b)
        _start(b)

    def body(k, carry):
        for b in range(NBUF):
            _wait(b)
            _unpack_dst(k * NBUF + b, b)
            _start(b)
        return carry

    lax.fori_loop(1, NGROUPS, body, 0)
    for b in range(NBUF):
        _wait(b)
    plsc.subcore_barrier()
    pltpu.sync_copy(acc_sh.at[pl.ds(r0, RLEN)], buf_v)
    pltpu.sync_copy(buf_v, out_hbm.at[pl.ds(cid * N + r0, RLEN)])


@functools.partial(
    pl.kernel,
    out_type=jax.ShapeDtypeStruct((N, NC * HID), jnp.float32),
    scratch_types=[
        pltpu.VMEM((EPT,), jnp.int32),
        pltpu.VMEM((NBUF, CHUNK), jnp.int32),
        pltpu.VMEM((NBUF, CHUNK), jnp.int32),
        pltpu.VMEM((NBUF, CHUNK, HID), jnp.float32),
        pltpu.VMEM((CHUNK, HID), jnp.float32),
        pltpu.VMEM_SHARED((N, HID), jnp.float32),
        pltpu.SemaphoreType.DMA,
        pltpu.SemaphoreType.DMA((NBUF,)),
        pltpu.SemaphoreType.DMA((NBUF,)),
    ],
    **_MESH,
)
def _sc_conv1(epk_hbm, g1_hbm, out_hbm,
              pk_b, sidx_p, didx_p, rows, zrow, acc_sh, zsem, gsem, ssem):
    cid = lax.axis_index("c")
    sid = lax.axis_index("s")
    t = cid * NS + sid
    r0 = sid * RSTRIDE
    pltpu.sync_copy(epk_hbm.at[pl.ds(t * EPT, EPT)], pk_b)
    _fill(zrow, (CHUNK, HID), 0.0)
    _zero_acc_slice(zrow, acc_sh, r0, zsem, CHUNK)
    plsc.subcore_barrier()

    def _unpack(c, b):
        for j in range(CHUNK // 16):
            pk = pk_b[pl.ds(c * CHUNK + j * 16, 16)]
            sidx_p[b, pl.ds(j * 16, 16)] = lax.shift_right_logical(pk, 14)
            didx_p[b, pl.ds(j * 16, 16)] = lax.bitwise_and(pk, 16383)

    def _gstart(b):
        pltpu.async_copy(g1_hbm.at[sidx_p.at[b]], rows.at[b], gsem.at[b])

    def _gwait(b):
        pltpu.make_async_copy(
            g1_hbm.at[sidx_p.at[b]], rows.at[b], gsem.at[b]).wait()

    def _sstart(b):
        pltpu.async_copy(
            rows.at[b], acc_sh.at[didx_p.at[b]], ssem.at[b], add=True)

    def _swait(b):
        pltpu.make_async_copy(
            rows.at[b], acc_sh.at[didx_p.at[b]], ssem.at[b]).wait()

    for b in range(NBUF):
        _unpack(b, b)
        _gstart(b)

    def body(k, carry):
        for b in range(NBUF):
            _gwait(b)
            _sstart(b)
        for b in range(NBUF):
            _swait(b)
            _unpack(k * NBUF + b, b)
            _gstart(b)
        return carry

    lax.fori_loop(1, NGROUPS, body, 0)
    for b in range(NBUF):
        _gwait(b)
        _sstart(b)
    for b in range(NBUF):
        _swait(b)
    plsc.subcore_barrier()
    c0 = cid * HID
    for j in range(RLEN // CHUNK):
        b = j % NBUF
        if j >= NBUF:
            pltpu.make_async_copy(
                rows.at[b],
                out_hbm.at[pl.ds(r0, CHUNK), pl.ds(c0, HID)],
                ssem.at[b]).wait()
        pltpu.async_copy(
            acc_sh.at[pl.ds(r0 + j * CHUNK, CHUNK)], rows.at[b], gsem.at[b])
        pltpu.make_async_copy(
            acc_sh.at[pl.ds(r0, CHUNK)], rows.at[b], gsem.at[b]).wait()
        pltpu.async_copy(
            rows.at[b],
            out_hbm.at[pl.ds(r0 + j * CHUNK, CHUNK), pl.ds(c0, HID)],
            ssem.at[b])
    for j in range(NBUF):
        pltpu.make_async_copy(
            rows.at[j], out_hbm.at[pl.ds(r0, CHUNK), pl.ds(c0, HID)],
            ssem.at[j]).wait()


@functools.partial(
    pl.kernel,
    out_type=jax.ShapeDtypeStruct((NC * N,), jnp.float32),
    scratch_types=[
        pltpu.VMEM((N,), jnp.float32),
        pltpu.VMEM((EPT,), jnp.int32),
        pltpu.VMEM((NBUF, CHUNK), jnp.int32),
        pltpu.VMEM((NBUF, CHUNK), jnp.float32),
        pltpu.VMEM((CHUNK,), jnp.float32),
        pltpu.VMEM((RLEN,), jnp.float32),
        pltpu.VMEM_SHARED((N,), jnp.float32),
        pltpu.SemaphoreType.DMA,
        pltpu.SemaphoreType.DMA((NBUF,)),
    ],
    **_MESH,
)
def _sc_conv2(epk_hbm, g2_hbm, out_hbm,
              g2_v, pk_b, didx_p, vals, zv, buf_v, acc_sh, zsem, ssem):
    cid = lax.axis_index("c")
    sid = lax.axis_index("s")
    t = cid * NS + sid
    r0 = sid * RSTRIDE
    pltpu.sync_copy(epk_hbm.at[pl.ds(t * EPT, EPT)], pk_b)
    pltpu.sync_copy(g2_hbm, g2_v)
    _fill(zv, (CHUNK,), 0.0)
    _zero_acc_slice(zv, acc_sh, r0, zsem, CHUNK)
    plsc.subcore_barrier()

    def _chunk(c, b):
        for j in range(CHUNK // 16):
            pk = pk_b[pl.ds(c * CHUNK + j * 16, 16)]
            sv = lax.shift_right_logical(pk, 14)
            didx_p[b, pl.ds(j * 16, 16)] = lax.bitwise_and(pk, 16383)
            vals[b, pl.ds(j * 16, 16)] = plsc.load_gather(g2_v, [sv])
        pltpu.async_copy(
            vals.at[b], acc_sh.at[didx_p.at[b]], ssem.at[b], add=True)

    def _swait(b):
        pltpu.make_async_copy(
            vals.at[b], acc_sh.at[didx_p.at[b]], ssem.at[b]).wait()

    for b in range(NBUF):
        _chunk(b, b)

    def body(k, carry):
        for b in range(NBUF):
            _swait(b)
            _chunk(k * NBUF + b, b)
        return carry

    lax.fori_loop(1, NGROUPS, body, 0)
    for b in range(NBUF):
        _swait(b)
    plsc.subcore_barrier()
    pltpu.sync_copy(acc_sh.at[pl.ds(r0, RLEN)], buf_v)
    pltpu.sync_copy(buf_v, out_hbm.at[pl.ds(cid * N + r0, RLEN)])


def _dense1_body(xt_ref, w1_ref, cnt_ref, g1_ref, dinv_ref):
    deg = cnt_ref[0] + cnt_ref[1] + 1.0
    dinv = lax.rsqrt(deg)
    h = lax.dot_general(xt_ref[...], w1_ref[...],
                        (((0,), (0,)), ((), ())),
                        preferred_element_type=jnp.float32)
    g1_ref[...] = h * dinv[:, None]
    dinv_ref[...] = dinv[:, None]


def _dense2_body(p_ref, g1_ref, dinv_ref, w2_ref, b1_ref, g2_ref):
    s1 = p_ref[:, pl.ds(0, HID)] + p_ref[:, pl.ds(HID, HID)] + g1_ref[...]
    dinv = dinv_ref[...]
    o1 = s1 * dinv + b1_ref[...][None, :]
    a = 0.5 * o1 * (1.0 + lax.erf(o1 * (1.0 / math.sqrt(2.0))))
    h2 = jnp.sum(a * w2_ref[:, 0][None, :], axis=1)
    g2_ref[...] = h2 * dinv[:, 0]


def _head_body(q_ref, g2_ref, dinv_ref, b2_ref, mw1_ref, mb1_ref,
               mw2_ref, mb2_ref, out_ref):
    s2 = q_ref[0] + q_ref[1] + g2_ref[...]
    o2 = s2[:, None] * dinv_ref[...] + b2_ref[0]
    t = o2 * mw1_ref[...] + mb1_ref[...][None, :]
    r = jnp.maximum(t, 0.0)
    y = jnp.sum(r * mw2_ref[:, 0][None, :], axis=1) + mb2_ref[0]
    out_ref[...] = y[:, None]


_dense1 = pl.pallas_call(
    _dense1_body,
    out_shape=[
        jax.ShapeDtypeStruct((N, HID), jnp.float32),
        jax.ShapeDtypeStruct((N, 1), jnp.float32),
    ],
)

_dense2 = pl.pallas_call(
    _dense2_body,
    out_shape=jax.ShapeDtypeStruct((N,), jnp.float32),
)

_head = pl.pallas_call(
    _head_body,
    out_shape=jax.ShapeDtypeStruct((N, 1), jnp.float32),
)


def kernel(x, edge_index, W1, b1, W2, b2, mW1, mb1, mW2, mb2):
    xt = x.reshape(N, IN_DIM).T
    epk = jnp.left_shift(edge_index[0], 14) | edge_index[1]

    cnt = _sc_deg(epk).reshape(NC, N)
    g1, dinv = _dense1(xt, W1, cnt)
    p = _sc_conv1(epk, g1)
    g2 = _dense2(p, g1, dinv, W2, b1)
    q = _sc_conv2(epk, g2).reshape(NC, N)
    y = _head(q, g2, dinv, b2, mW1, mb1, mW2, mb2)
    return y.reshape(1, 1, N, 1)
